# Initial kernel scaffold; baseline (speedup 1.0000x reference)
import functools

import jax
import jax.numpy as jnp
from jax import lax
from jax.experimental import pallas as pl
from jax.experimental.pallas import tpu as pltpu
from jax.experimental.pallas import tpu_sc as plsc

_N = 10000
_E = 320000
_IN = 128
_HID = 256
_OUT = 128

_NC = 2
_NS = 16
_L = 16
_NW = _NC * _NS
_EB = 128
_EPW = -(-_E // (_NW * _EB)) * _EB
_E_PAD = _EPW * _NW
_NROWS = 10240
_RPT = _NROWS // _NS

_ROWBLK = 400


def _seg_sum_sc(feat, src, dst, with_deg):
    d = feat.shape[1]
    mesh = plsc.VectorSubcoreMesh(core_axis_name="c", subcore_axis_name="s")
    out_type = [jax.ShapeDtypeStruct((_NC, _NROWS, d), jnp.float32)]
    scratch = [
        pltpu.VMEM((_EB,), jnp.int32),
        pltpu.VMEM((_EB,), jnp.int32),
        pltpu.VMEM((_EB, d), jnp.float32),
        pltpu.VMEM((8, d), jnp.float32),
        pltpu.VMEM_SHARED((_NROWS, d), jnp.float32),
        pltpu.SemaphoreType.DMA,
    ]
    if with_deg:
        out_type.append(jax.ShapeDtypeStruct((_NC, _NROWS, _L), jnp.float32))
        scratch += [
            pltpu.VMEM((_EB, _L), jnp.float32),
            pltpu.VMEM((8, _L), jnp.float32),
            pltpu.VMEM_SHARED((_NROWS, _L), jnp.float32),
        ]

    @functools.partial(pl.kernel, mesh=mesh, out_type=out_type,
                       scratch_types=scratch)
    def k(feat_hbm, src_hbm, dst_hbm, out_hbm, *rest):
        if with_deg:
            (deg_hbm, sidx, didx, rows, zrow, acc, sem,
             ones, zdeg, dacc) = rest
        else:
            sidx, didx, rows, zrow, acc, sem = rest
        cid = lax.axis_index("c")
        sid = lax.axis_index("s")
        wid = cid * _NS + sid

        for i in range(8):
            for j in range(d // _L):
                zrow[i, pl.ds(j * _L, _L)] = jnp.zeros((_L,), jnp.float32)
        if with_deg:
            for i in range(8):
                zdeg[i, pl.ds(0, _L)] = jnp.zeros((_L,), jnp.float32)

            @pl.loop(0, _EB)
            def _(i):
                ones[i, pl.ds(0, _L)] = jnp.ones((_L,), jnp.float32)

        @pl.loop(0, _RPT, step=8)
        def _(r):
            pltpu.sync_copy(zrow, acc.at[pl.ds(sid * _RPT + r, 8)])

        if with_deg:
            @pl.loop(0, _RPT, step=8)
            def _(r):
                pltpu.sync_copy(zdeg, dacc.at[pl.ds(sid * _RPT + r, 8)])

        plsc.subcore_barrier()

        base = wid * _EPW

        @pl.loop(0, _EPW, step=_EB)
        def _(e):
            pltpu.sync_copy(src_hbm.at[pl.ds(base + e, _EB)], sidx)
            pltpu.sync_copy(dst_hbm.at[pl.ds(base + e, _EB)], didx)
            pltpu.async_copy(feat_hbm.at[sidx], rows, sem).wait()
            pltpu.sync_copy(rows, acc.at[didx], add=True)
            if with_deg:
                pltpu.sync_copy(ones, dacc.at[didx], add=True)

        plsc.subcore_barrier()

        rs = pl.ds(sid * _RPT, _RPT)
        pltpu.sync_copy(acc.at[rs], out_hbm.at[cid].at[rs])
        if with_deg:
            pltpu.sync_copy(dacc.at[rs], deg_hbm.at[cid].at[rs])

    return k(feat, src, dst)


def _gelu(h):
    return 0.5 * h * (1.0 + lax.erf(h * 0.7071067811865476))


def _layer1_body(x_ref, s1_ref, deg_ref, w1l_ref, w1r_ref, b1_ref,
                 w2l_ref, w2r_ref, b2_ref, p_ref, q_ref):
    deg = deg_ref[0, :, 0:1] + deg_ref[1, :, 0:1]
    agg = (s1_ref[0] + s1_ref[1]) / jnp.maximum(deg, 1.0)
    h = (jnp.dot(agg, w1l_ref[...], preferred_element_type=jnp.float32)
         + jnp.dot(x_ref[...], w1r_ref[...], preferred_element_type=jnp.float32)
         + b1_ref[...])
    h = _gelu(h)
    p_ref[...] = jnp.dot(h, w2l_ref[...], preferred_element_type=jnp.float32)
    q_ref[...] = (jnp.dot(h, w2r_ref[...], preferred_element_type=jnp.float32)
                  + b2_ref[...])


def _layer2_body(s2_ref, deg_ref, q_ref, out_ref):
    deg = deg_ref[0, :, 0:1] + deg_ref[1, :, 0:1]
    out_ref[...] = (s2_ref[0] + s2_ref[1]) / jnp.maximum(deg, 1.0) + q_ref[...]


def kernel(x, edge_index, W1_l, W1_r, b1, W2_l, W2_r, b2):
    src = edge_index[0].astype(jnp.int32)
    dst = edge_index[1].astype(jnp.int32)
    pad = _E_PAD - _E
    src = jnp.concatenate([src, jnp.zeros((pad,), jnp.int32)])
    dst = jnp.concatenate([dst, jnp.full((pad,), _N, jnp.int32)])

    sum1, deg = _seg_sum_sc(x, src, dst, with_deg=True)

    nblk = _N // _ROWBLK
    b1r = b1.reshape(1, _HID)
    b2r = b2.reshape(1, _OUT)
    p, q = pl.pallas_call(
        _layer1_body,
        grid=(nblk,),
        in_specs=[
            pl.BlockSpec((_ROWBLK, _IN), lambda i: (i, 0)),
            pl.BlockSpec((_NC, _ROWBLK, _IN), lambda i: (0, i, 0)),
            pl.BlockSpec((_NC, _ROWBLK, _L), lambda i: (0, i, 0)),
            pl.BlockSpec((_IN, _HID), lambda i: (0, 0)),
            pl.BlockSpec((_IN, _HID), lambda i: (0, 0)),
            pl.BlockSpec((1, _HID), lambda i: (0, 0)),
            pl.BlockSpec((_HID, _OUT), lambda i: (0, 0)),
            pl.BlockSpec((_HID, _OUT), lambda i: (0, 0)),
            pl.BlockSpec((1, _OUT), lambda i: (0, 0)),
        ],
        out_specs=[
            pl.BlockSpec((_ROWBLK, _OUT), lambda i: (i, 0)),
            pl.BlockSpec((_ROWBLK, _OUT), lambda i: (i, 0)),
        ],
        out_shape=[
            jax.ShapeDtypeStruct((_N, _OUT), jnp.float32),
            jax.ShapeDtypeStruct((_N, _OUT), jnp.float32),
        ],
    )(x, sum1, deg, W1_l, W1_r, b1r, W2_l, W2_r, b2r)

    (sum2,) = _seg_sum_sc(p, src, dst, with_deg=False)

    out = pl.pallas_call(
        _layer2_body,
        grid=(nblk,),
        in_specs=[
            pl.BlockSpec((_NC, _ROWBLK, _OUT), lambda i: (0, i, 0)),
            pl.BlockSpec((_NC, _ROWBLK, _L), lambda i: (0, i, 0)),
            pl.BlockSpec((_ROWBLK, _OUT), lambda i: (i, 0)),
        ],
        out_specs=pl.BlockSpec((_ROWBLK, _OUT), lambda i: (i, 0)),
        out_shape=jax.ShapeDtypeStruct((_N, _OUT), jnp.float32),
    )(sum2, deg, q)
    return out


# speedup vs baseline: 5.0466x; 5.0466x over previous
import dataclasses
import functools

import jax
import jax.numpy as jnp
from jax import lax
from jax.experimental import pallas as pl
from jax.experimental.pallas import tpu as pltpu
from jax.experimental.pallas import tpu_sc as plsc

_N = 10000
_E = 320000
_IN = 128
_HID = 256
_OUT = 128

_NC = 2
_NS = 16
_L = 16
_NW = _NC * _NS
_EB = 128
_EPW = -(-_E // (_NW * _EB)) * _EB
_E_PAD = _EPW * _NW
_NROWS = 10240
_RPT = _NROWS // _NS

_ROWBLK = 400


def _seg_sum_sc(feat, src, dst, with_deg, edge_loop=True):
    d = feat.shape[1]
    mesh = plsc.VectorSubcoreMesh(core_axis_name="c", subcore_axis_name="s")
    out_type = [jax.ShapeDtypeStruct((_NC, _NROWS, d), jnp.float32)]
    scratch = [
        pltpu.VMEM((_EB,), jnp.int32),
        pltpu.VMEM((_EB,), jnp.int32),
        pltpu.VMEM((_EB, d), jnp.float32),
        pltpu.VMEM((8, d), jnp.float32),
        pltpu.VMEM_SHARED((_NROWS, d), jnp.float32),
        pltpu.SemaphoreType.DMA,
    ]
    if with_deg:
        out_type.append(jax.ShapeDtypeStruct((_NW, _NROWS), jnp.float32))
        scratch += [
            pltpu.VMEM((_NROWS,), jnp.float32),
        ]

    kw = {}
    if "needs_layout_passes" in pltpu.CompilerParams.__dataclass_fields__:
        kw["compiler_params"] = dataclasses.replace(
            pltpu.CompilerParams(), needs_layout_passes=False)

    @functools.partial(pl.kernel, mesh=mesh, out_type=out_type,
                       scratch_types=scratch, **kw)
    def k(feat_hbm, src_hbm, dst_hbm, out_hbm, *rest):
        if with_deg:
            deg_hbm, sidx, didx, rows, zrow, acc, sem, hist = rest
        else:
            sidx, didx, rows, zrow, acc, sem = rest
        cid = lax.axis_index("c")
        sid = lax.axis_index("s")
        wid = cid * _NS + sid

        for i in range(8):
            for j in range(d // _L):
                zrow[i, pl.ds(j * _L, _L)] = jnp.zeros((_L,), jnp.float32)
        if with_deg:
            @pl.loop(0, _NROWS, step=_L)
            def _(r):
                hist[pl.ds(r, _L)] = jnp.zeros((_L,), jnp.float32)

        @pl.loop(0, _RPT, step=8)
        def _(r):
            pltpu.sync_copy(zrow, acc.at[pl.ds(sid * _RPT + r, 8)])

        plsc.subcore_barrier()

        base = wid * _EPW

        if edge_loop:
            @pl.loop(0, _EPW, step=_EB)
            def _(e):
                pltpu.sync_copy(src_hbm.at[pl.ds(base + e, _EB)], sidx)
                pltpu.sync_copy(dst_hbm.at[pl.ds(base + e, _EB)], didx)
                pltpu.async_copy(feat_hbm.at[sidx], rows, sem).wait()
                pltpu.sync_copy(rows, acc.at[didx], add=True)
                if with_deg:
                    ones16 = jnp.ones((_L,), jnp.float32)
                    for kk in range(_EB // _L):
                        idxr = didx[pl.ds(kk * _L, _L)]
                        plsc.addupdate_scatter(hist, [idxr], ones16)

        plsc.subcore_barrier()

        rs = pl.ds(sid * _RPT, _RPT)
        pltpu.sync_copy(acc.at[rs], out_hbm.at[cid, rs])
        if with_deg:
            pltpu.sync_copy(hist, deg_hbm.at[wid])

    res = k(feat, src, dst)
    if with_deg:
        return res[0], res[1]
    return res[0] if isinstance(res, (list, tuple)) else res


def _gelu(h):
    return 0.5 * h * (1.0 + lax.erf(h * 0.7071067811865476))


def _layer1_body(x_ref, s1a_ref, s1b_ref, deg_ref, w1l_ref, w1r_ref, b1_ref,
                 w2l_ref, w2r_ref, b2_ref, p_ref, q_ref):
    deg = jnp.sum(deg_ref[...], axis=1)[:, None]
    agg = (s1a_ref[...] + s1b_ref[...]) / jnp.maximum(deg, 1.0)
    h = (jnp.dot(agg, w1l_ref[...], preferred_element_type=jnp.float32)
         + jnp.dot(x_ref[...], w1r_ref[...], preferred_element_type=jnp.float32)
         + b1_ref[...])
    h = _gelu(h)
    p_ref[...] = jnp.dot(h, w2l_ref[...], preferred_element_type=jnp.float32)
    q_ref[...] = (jnp.dot(h, w2r_ref[...], preferred_element_type=jnp.float32)
                  + b2_ref[...])


def _layer2_body(s2a_ref, s2b_ref, deg_ref, q_ref, out_ref):
    deg = jnp.sum(deg_ref[...], axis=1)[:, None]
    out_ref[...] = ((s2a_ref[...] + s2b_ref[...]) / jnp.maximum(deg, 1.0)
                    + q_ref[...])


def kernel(x, edge_index, W1_l, W1_r, b1, W2_l, W2_r, b2):
    src = edge_index[0].astype(jnp.int32)
    dst = edge_index[1].astype(jnp.int32)
    pad = _E_PAD - _E
    src = jnp.concatenate([src, jnp.zeros((pad,), jnp.int32)])
    dst = jnp.concatenate([dst, jnp.full((pad,), _N, jnp.int32)])

    sum1, deg = _seg_sum_sc(x, src, dst, with_deg=True)
    degT = deg.T

    nblk = _N // _ROWBLK
    b1r = b1.reshape(1, _HID)
    b2r = b2.reshape(1, _OUT)
    p, q = pl.pallas_call(
        _layer1_body,
        grid=(nblk,),
        in_specs=[
            pl.BlockSpec((_ROWBLK, _IN), lambda i: (i, 0)),
            pl.BlockSpec((_ROWBLK, _IN), lambda i: (i, 0)),
            pl.BlockSpec((_ROWBLK, _IN), lambda i: (i, 0)),
            pl.BlockSpec((_ROWBLK, _NW), lambda i: (i, 0)),
            pl.BlockSpec((_IN, _HID), lambda i: (0, 0)),
            pl.BlockSpec((_IN, _HID), lambda i: (0, 0)),
            pl.BlockSpec((1, _HID), lambda i: (0, 0)),
            pl.BlockSpec((_HID, _OUT), lambda i: (0, 0)),
            pl.BlockSpec((_HID, _OUT), lambda i: (0, 0)),
            pl.BlockSpec((1, _OUT), lambda i: (0, 0)),
        ],
        out_specs=[
            pl.BlockSpec((_ROWBLK, _OUT), lambda i: (i, 0)),
            pl.BlockSpec((_ROWBLK, _OUT), lambda i: (i, 0)),
        ],
        out_shape=[
            jax.ShapeDtypeStruct((_N, _OUT), jnp.float32),
            jax.ShapeDtypeStruct((_N, _OUT), jnp.float32),
        ],
    )(x, sum1[0], sum1[1], degT, W1_l, W1_r, b1r, W2_l, W2_r, b2r)

    sum2 = _seg_sum_sc(p, src, dst, with_deg=False)

    out = pl.pallas_call(
        _layer2_body,
        grid=(nblk,),
        in_specs=[
            pl.BlockSpec((_ROWBLK, _OUT), lambda i: (i, 0)),
            pl.BlockSpec((_ROWBLK, _OUT), lambda i: (i, 0)),
            pl.BlockSpec((_ROWBLK, _NW), lambda i: (i, 0)),
            pl.BlockSpec((_ROWBLK, _OUT), lambda i: (i, 0)),
        ],
        out_specs=pl.BlockSpec((_ROWBLK, _OUT), lambda i: (i, 0)),
        out_shape=jax.ShapeDtypeStruct((_N, _OUT), jnp.float32),
    )(sum2[0], sum2[1], degT, q)
    return out
